# baseline (speedup 1.0000x reference)
import functools

import jax
import jax.numpy as jnp
from jax import lax
from jax.experimental import pallas as pl
from jax.experimental.pallas import tpu as pltpu
from jax.experimental.pallas import tpu_sc as plsc

_info = plsc.get_sparse_core_info()
_NC, _NS = _info.num_cores, _info.num_subcores
_NW = _NC * _NS

_VOCAB = 1000000
_D = 32
_B_TOTAL = 4096 * 200
_B_PER_W = _B_TOTAL // _NW
_CH = 640
_NCH = _B_PER_W // _CH
_NBUF = 4

_mesh = plsc.VectorSubcoreMesh(core_axis_name="c", subcore_axis_name="s")


@functools.partial(
    pl.kernel,
    out_type=jax.ShapeDtypeStruct((_B_TOTAL, _D), jnp.float32),
    mesh=_mesh,
    scratch_types=[
        pltpu.VMEM((_B_PER_W,), jnp.int32),
        [pltpu.VMEM((_CH, _D), jnp.float32) for _ in range(_NBUF)],
        [pltpu.SemaphoreType.DMA for _ in range(_NBUF)],
        [pltpu.SemaphoreType.DMA for _ in range(_NBUF)],
    ],
    compiler_params=pltpu.CompilerParams(use_tc_tiling_on_sc=False),
)
def _gather_kernel(idx_hbm, table_hbm, out_hbm, idx_v, rows, gsem, wsem):
    wid = lax.axis_index("s") * _NC + lax.axis_index("c")
    base = wid * _B_PER_W

    pltpu.sync_copy(idx_hbm.at[pl.ds(base, _B_PER_W)], idx_v)

    def gather(c, b):
        return pltpu.async_copy(
            table_hbm.at[idx_v.at[pl.ds(c * _CH, _CH)]], rows[b], gsem[b])

    def gather_wait(b):
        pltpu.make_async_copy(
            table_hbm.at[idx_v.at[pl.ds(0, _CH)]], rows[b], gsem[b]).wait()

    def write(c, b):
        return pltpu.async_copy(
            rows[b], out_hbm.at[pl.ds(base + c * _CH, _CH)], wsem[b])

    def write_wait(b):
        pltpu.make_async_copy(
            rows[b], out_hbm.at[pl.ds(base, _CH)], wsem[b]).wait()

    for b in range(_NBUF):
        gather(b, b)

    @pl.loop(0, _NCH, step=_NBUF)
    def _(c):
        for b in range(_NBUF):
            g = c + b
            gather_wait(b)
            write(g, b)

            @pl.when(g + _NBUF < _NCH)
            def _():
                write_wait(b)
                gather(g + _NBUF, b)

    for b in range(_NBUF):
        write_wait(b)


def kernel(x, table):
    out = _gather_kernel(x.reshape(-1), table)
    return out.reshape(x.shape[0], x.shape[1], _D)
